# baseline (speedup 1.0000x reference)
import functools

import jax
import jax.numpy as jnp
from jax import lax
from jax.experimental import pallas as pl
from jax.experimental.pallas import tpu as pltpu
from jax.experimental.pallas import tpu_sc as plsc

N = 4096
D = 768
E = 16
FF = 1024
K = 2

T = 256
NT = (N * K) // T + E
P = NT * T
RB = 512
NB = N // RB

NW = 32
TPW = N // NW
SPW = P // NW
GC = 32


def _router_body(x_ref, wr_ref, br_ref, route_ref, slots_ref, te_ref,
                 carry, rbuf):
    i = pl.program_id(0)

    @pl.when(i == 0)
    def _init():
        carry[...] = jnp.zeros((1, E), jnp.float32)

    x = x_ref[...]
    logits = jnp.dot(x, wr_ref[...], preferred_element_type=jnp.float32)
    logits = logits + br_ref[...]
    lane = lax.broadcasted_iota(jnp.int32, (RB, E), 1)
    m1 = jnp.max(logits, axis=1, keepdims=True)
    idx1 = jnp.min(jnp.where(logits == m1, lane, E), axis=1, keepdims=True)
    l2 = jnp.where(lane == idx1, -jnp.inf, logits)
    m2 = jnp.max(l2, axis=1, keepdims=True)
    idx2 = jnp.min(jnp.where(l2 == m2, lane, E), axis=1, keepdims=True)
    b = jnp.exp(m2 - m1)
    w0 = 1.0 / (1.0 + b)
    w1 = b / (1.0 + b)

    oh0 = (lane == idx1).astype(jnp.float32)
    oh1 = (lane == idx2).astype(jnp.float32)
    ri = lax.broadcasted_iota(jnp.int32, (RB, RB), 0)
    ci = lax.broadcasted_iota(jnp.int32, (RB, RB), 1)
    ts = (ci < ri).astype(jnp.float32)
    c0 = carry[...]
    p0 = jnp.dot(ts, oh0, preferred_element_type=jnp.float32) + c0
    rank0 = jnp.sum(p0 * oh0, axis=1, keepdims=True)
    c1 = c0 + jnp.sum(oh0, axis=0, keepdims=True)
    p1 = jnp.dot(ts, oh1, preferred_element_type=jnp.float32) + c1
    rank1 = jnp.sum(p1 * oh1, axis=1, keepdims=True)
    c2 = c1 + jnp.sum(oh1, axis=0, keepdims=True)
    carry[...] = c2

    col = lax.broadcasted_iota(jnp.int32, (RB, 8), 1)
    route = jnp.where(col == 0, idx1.astype(jnp.float32), 0.0)
    route = route + jnp.where(col == 1, idx2.astype(jnp.float32), 0.0)
    route = route + jnp.where(col == 2, w0, 0.0)
    route = route + jnp.where(col == 3, w1, 0.0)
    route = route + jnp.where(col == 4, rank0, 0.0)
    route = route + jnp.where(col == 5, rank1, 0.0)
    route_ref[...] = route
    rbuf[pl.ds(i * RB, RB), :] = route

    @pl.when(i == NB - 1)
    def _finish():
        padded = jnp.floor((c2 + (T - 1)) * (1.0 / T)) * T
        ue = lax.broadcasted_iota(jnp.int32, (E, E), 0)
        uc = lax.broadcasted_iota(jnp.int32, (E, E), 1)
        ustrict = (ue < uc).astype(jnp.float32)
        base = jnp.dot(padded, ustrict,
                       preferred_element_type=jnp.float32)

        ti = lax.broadcasted_iota(jnp.int32, (1, 128), 1).astype(jnp.float32) * T
        acc = jnp.zeros((1, 128), jnp.float32)
        for e in range(E):
            acc = acc + (ti >= base[0:1, e:e + 1]).astype(jnp.float32)
        pend = base[0:1, E - 1:E] + padded[0:1, E - 1:E]
        act = (ti < pend).astype(jnp.float32)
        rowi = lax.broadcasted_iota(jnp.int32, (2, 128), 0)
        te_ref[...] = jnp.where(rowi == 0, acc - 1.0, act)

        colf = lax.broadcasted_iota(jnp.int32, (RB, 8), 1)
        col2 = lax.broadcasted_iota(jnp.int32, (RB, K), 1)
        lanef = lax.broadcasted_iota(jnp.int32, (RB, E), 1).astype(jnp.float32)
        for j in range(NB):
            r = rbuf[pl.ds(j * RB, RB), :]
            key0 = jnp.sum(jnp.where(colf == 0, r, 0.0), axis=1, keepdims=True)
            key1 = jnp.sum(jnp.where(colf == 1, r, 0.0), axis=1, keepdims=True)
            rk0 = jnp.sum(jnp.where(colf == 4, r, 0.0), axis=1, keepdims=True)
            rk1 = jnp.sum(jnp.where(colf == 5, r, 0.0), axis=1, keepdims=True)
            acc0 = jnp.sum(jnp.where(lanef == key0, base, 0.0),
                           axis=1, keepdims=True)
            acc1 = jnp.sum(jnp.where(lanef == key1, base, 0.0),
                           axis=1, keepdims=True)
            slots_ref[pl.ds(j * RB, RB), :] = jnp.where(
                col2 == 0, acc0 + rk0, acc1 + rk1)


def _router(x, Wr, br):
    return pl.pallas_call(
        _router_body,
        grid=(NB,),
        in_specs=[
            pl.BlockSpec((RB, D), lambda i: (i, 0)),
            pl.BlockSpec((D, E), lambda i: (0, 0)),
            pl.BlockSpec((1, E), lambda i: (0, 0)),
        ],
        out_specs=[
            pl.BlockSpec((RB, 8), lambda i: (i, 0)),
            pl.BlockSpec((N, K), lambda i: (0, 0)),
            pl.BlockSpec((2, 128), lambda i: (0, 0)),
        ],
        out_shape=[
            jax.ShapeDtypeStruct((N, 8), jnp.float32),
            jax.ShapeDtypeStruct((N, K), jnp.float32),
            jax.ShapeDtypeStruct((2, 128), jnp.float32),
        ],
        scratch_shapes=[pltpu.VMEM((1, E), jnp.float32),
                        pltpu.VMEM((N, 8), jnp.float32)],
    )(x, Wr, br.reshape(1, E))


def _sc_scatter_x_body(slot0_hbm, slot1_hbm, x_hbm, xg_hbm,
                       xv, slots_v, sem0, sem1):
    wid = lax.axis_index("s") * 2 + lax.axis_index("c")
    t0 = wid * TPW
    pltpu.sync_copy(slot0_hbm.at[pl.ds(t0, TPW)], slots_v.at[0])
    pltpu.sync_copy(slot1_hbm.at[pl.ds(t0, TPW)], slots_v.at[1])
    pltpu.sync_copy(x_hbm.at[pl.ds(t0, TPW)], xv)
    cp0 = pltpu.async_copy(xv, xg_hbm.at[slots_v.at[0]], sem0)
    cp1 = pltpu.async_copy(xv, xg_hbm.at[slots_v.at[1]], sem1)
    cp0.wait()
    cp1.wait()


NCHK = TPW // GC


def _sc_gather_y_body(slot0_hbm, slot1_hbm, yg_hbm, y0_hbm, y1_hbm,
                      slots_v, y0v, y1v, sem00, sem01, sem10, sem11):
    wid = lax.axis_index("s") * 2 + lax.axis_index("c")
    t0 = wid * TPW
    pltpu.sync_copy(slot0_hbm.at[pl.ds(t0, TPW)], slots_v.at[0])
    pltpu.sync_copy(slot1_hbm.at[pl.ds(t0, TPW)], slots_v.at[1])
    sems0 = [sem00, sem01]
    sems1 = [sem10, sem11]

    def fire(c, b):
        cp0 = pltpu.async_copy(
            yg_hbm.at[slots_v.at[0, pl.ds(c * GC, GC)]], y0v.at[b], sems0[b])
        cp1 = pltpu.async_copy(
            yg_hbm.at[slots_v.at[1, pl.ds(c * GC, GC)]], y1v.at[b], sems1[b])
        return cp0, cp1

    pend = {}
    pend[0] = fire(0, 0)
    if NCHK > 1:
        pend[1] = fire(1, 1)
    for c in range(NCHK):
        b = c % 2
        cp0, cp1 = pend.pop(c)
        cp0.wait()
        cp1.wait()
        pltpu.sync_copy(y0v.at[b], y0_hbm.at[pl.ds(t0 + c * GC, GC)])
        pltpu.sync_copy(y1v.at[b], y1_hbm.at[pl.ds(t0 + c * GC, GC)])
        if c + 2 < NCHK:
            pend[c + 2] = fire(c + 2, b)


@functools.lru_cache(maxsize=None)
def _sc_kernels():
    mesh = plsc.VectorSubcoreMesh(core_axis_name="c", subcore_axis_name="s")
    scatter_x = functools.partial(
        pl.kernel, mesh=mesh,
        out_type=jax.ShapeDtypeStruct((P, D), jnp.float32),
        scratch_types=[
            pltpu.VMEM((TPW, D), jnp.float32),
            pltpu.VMEM((K, TPW), jnp.int32),
            pltpu.SemaphoreType.DMA,
            pltpu.SemaphoreType.DMA,
        ],
    )(_sc_scatter_x_body)
    gather_y = functools.partial(
        pl.kernel, mesh=mesh,
        out_type=[jax.ShapeDtypeStruct((N, D), jnp.float32),
                  jax.ShapeDtypeStruct((N, D), jnp.float32)],
        scratch_types=[
            pltpu.VMEM((K, TPW), jnp.int32),
            pltpu.VMEM((2, GC, D), jnp.float32),
            pltpu.VMEM((2, GC, D), jnp.float32),
            pltpu.SemaphoreType.DMA,
            pltpu.SemaphoreType.DMA,
            pltpu.SemaphoreType.DMA,
            pltpu.SemaphoreType.DMA,
        ],
    )(_sc_gather_y_body)
    return scatter_x, gather_y


def _ffn_body(te_ref, act_ref, xg_ref, w1_ref, b1_ref, w2_ref, b2_ref, yg_ref):
    i = pl.program_id(0)

    @pl.when(act_ref[i] == 1)
    def _compute():
        h = jnp.dot(xg_ref[...], w1_ref[0], preferred_element_type=jnp.float32)
        h = jnp.maximum(h + b1_ref[0], 0.0)
        y = jnp.dot(h, w2_ref[0], preferred_element_type=jnp.float32)
        yg_ref[...] = y + b2_ref[0]


def _grouped_ffn(te, act, xg, W1, b1, W2, b2):
    grid_spec = pltpu.PrefetchScalarGridSpec(
        num_scalar_prefetch=2,
        grid=(NT,),
        in_specs=[
            pl.BlockSpec((T, D), lambda i, te, act: (i, 0)),
            pl.BlockSpec((1, D, FF), lambda i, te, act: (te[i], 0, 0)),
            pl.BlockSpec((1, 1, FF), lambda i, te, act: (te[i], 0, 0)),
            pl.BlockSpec((1, FF, D), lambda i, te, act: (te[i], 0, 0)),
            pl.BlockSpec((1, 1, D), lambda i, te, act: (te[i], 0, 0)),
        ],
        out_specs=pl.BlockSpec((T, D), lambda i, te, act: (i, 0)),
    )
    return pl.pallas_call(
        _ffn_body,
        grid_spec=grid_spec,
        out_shape=jax.ShapeDtypeStruct((P, D), jnp.float32),
    )(te, act, xg, W1, b1.reshape(E, 1, FF), W2, b2.reshape(E, 1, D))


CB = 1024


def _combine_body(x_ref, r_ref, y0_ref, y1_ref, out_ref):
    col = lax.broadcasted_iota(jnp.int32, (CB, 8), 1)
    r = r_ref[...]
    w0 = jnp.sum(jnp.where(col == 2, r, 0.0), axis=1, keepdims=True)
    w1 = jnp.sum(jnp.where(col == 3, r, 0.0), axis=1, keepdims=True)
    out_ref[...] = x_ref[...] + w0 * y0_ref[...] + w1 * y1_ref[...]


def _combine(x, route, y0, y1):
    return pl.pallas_call(
        _combine_body,
        grid=(N // CB,),
        in_specs=[
            pl.BlockSpec((CB, D), lambda i: (i, 0)),
            pl.BlockSpec((CB, 8), lambda i: (i, 0)),
            pl.BlockSpec((CB, D), lambda i: (i, 0)),
            pl.BlockSpec((CB, D), lambda i: (i, 0)),
        ],
        out_specs=pl.BlockSpec((CB, D), lambda i: (i, 0)),
        out_shape=jax.ShapeDtypeStruct((N, D), jnp.float32),
    )(x, route, y0, y1)


def kernel(x, Wr, br, W1, b1, W2, b2):
    sc_scatter_x, sc_gather_y = _sc_kernels()
    route, slots, te_f = _router(x, Wr, br)
    te = te_f[0, :NT].astype(jnp.int32)
    act = te_f[1, :NT].astype(jnp.int32)
    st = slots.astype(jnp.int32).T
    slot0 = st[0]
    slot1 = st[1]
    xg = sc_scatter_x(slot0, slot1, x)
    yg = _grouped_ffn(te, act, xg, W1, b1, W2, b2)
    y0, y1 = sc_gather_y(slot0, slot1, yg)
    return _combine(x, route, y0, y1)


# speedup vs baseline: 1.0651x; 1.0651x over previous
import functools

import jax
import jax.numpy as jnp
from jax import lax
from jax.experimental import pallas as pl
from jax.experimental.pallas import tpu as pltpu
from jax.experimental.pallas import tpu_sc as plsc

N = 4096
D = 768
E = 16
FF = 1024
K = 2

T = 512
NT = (N * K) // T + E
P = NT * T
RB = 512
NB = N // RB

NW = 32
TPW = N // NW
SPW = P // NW
GC = 32


def _router_body(x_ref, wr_ref, br_ref, route_ref, slots_ref, te_ref,
                 carry, rbuf):
    i = pl.program_id(0)

    @pl.when(i == 0)
    def _init():
        carry[...] = jnp.zeros((1, E), jnp.float32)

    x = x_ref[...]
    logits = jnp.dot(x, wr_ref[...], preferred_element_type=jnp.float32)
    logits = logits + br_ref[...]
    lane = lax.broadcasted_iota(jnp.int32, (RB, E), 1)
    m1 = jnp.max(logits, axis=1, keepdims=True)
    idx1 = jnp.min(jnp.where(logits == m1, lane, E), axis=1, keepdims=True)
    l2 = jnp.where(lane == idx1, -jnp.inf, logits)
    m2 = jnp.max(l2, axis=1, keepdims=True)
    idx2 = jnp.min(jnp.where(l2 == m2, lane, E), axis=1, keepdims=True)
    b = jnp.exp(m2 - m1)
    w0 = 1.0 / (1.0 + b)
    w1 = b / (1.0 + b)

    oh0 = (lane == idx1).astype(jnp.float32)
    oh1 = (lane == idx2).astype(jnp.float32)
    ri = lax.broadcasted_iota(jnp.int32, (RB, RB), 0)
    ci = lax.broadcasted_iota(jnp.int32, (RB, RB), 1)
    ts = (ci < ri).astype(jnp.float32)
    c0 = carry[...]
    p0 = jnp.dot(ts, oh0, preferred_element_type=jnp.float32) + c0
    rank0 = jnp.sum(p0 * oh0, axis=1, keepdims=True)
    c1 = c0 + jnp.sum(oh0, axis=0, keepdims=True)
    p1 = jnp.dot(ts, oh1, preferred_element_type=jnp.float32) + c1
    rank1 = jnp.sum(p1 * oh1, axis=1, keepdims=True)
    c2 = c1 + jnp.sum(oh1, axis=0, keepdims=True)
    carry[...] = c2

    col = lax.broadcasted_iota(jnp.int32, (RB, 8), 1)
    route = jnp.where(col == 0, idx1.astype(jnp.float32), 0.0)
    route = route + jnp.where(col == 1, idx2.astype(jnp.float32), 0.0)
    route = route + jnp.where(col == 2, w0, 0.0)
    route = route + jnp.where(col == 3, w1, 0.0)
    route = route + jnp.where(col == 4, rank0, 0.0)
    route = route + jnp.where(col == 5, rank1, 0.0)
    route_ref[...] = route
    rbuf[pl.ds(i * RB, RB), :] = route

    @pl.when(i == NB - 1)
    def _finish():
        padded = jnp.floor((c2 + (T - 1)) * (1.0 / T)) * T
        ue = lax.broadcasted_iota(jnp.int32, (E, E), 0)
        uc = lax.broadcasted_iota(jnp.int32, (E, E), 1)
        ustrict = (ue < uc).astype(jnp.float32)
        base = jnp.dot(padded, ustrict,
                       preferred_element_type=jnp.float32)

        ti = lax.broadcasted_iota(jnp.int32, (1, 128), 1).astype(jnp.float32) * T
        acc = jnp.zeros((1, 128), jnp.float32)
        for e in range(E):
            acc = acc + (ti >= base[0:1, e:e + 1]).astype(jnp.float32)
        pend = base[0:1, E - 1:E] + padded[0:1, E - 1:E]
        act = (ti < pend).astype(jnp.float32)
        rowi = lax.broadcasted_iota(jnp.int32, (2, 128), 0)
        te_ref[...] = jnp.where(rowi == 0, acc - 1.0, act)

        colf = lax.broadcasted_iota(jnp.int32, (RB, 8), 1)
        col2 = lax.broadcasted_iota(jnp.int32, (RB, K), 1)
        lanef = lax.broadcasted_iota(jnp.int32, (RB, E), 1).astype(jnp.float32)
        for j in range(NB):
            r = rbuf[pl.ds(j * RB, RB), :]
            key0 = jnp.sum(jnp.where(colf == 0, r, 0.0), axis=1, keepdims=True)
            key1 = jnp.sum(jnp.where(colf == 1, r, 0.0), axis=1, keepdims=True)
            rk0 = jnp.sum(jnp.where(colf == 4, r, 0.0), axis=1, keepdims=True)
            rk1 = jnp.sum(jnp.where(colf == 5, r, 0.0), axis=1, keepdims=True)
            acc0 = jnp.sum(jnp.where(lanef == key0, base, 0.0),
                           axis=1, keepdims=True)
            acc1 = jnp.sum(jnp.where(lanef == key1, base, 0.0),
                           axis=1, keepdims=True)
            slots_ref[pl.ds(j * RB, RB), :] = jnp.where(
                col2 == 0, acc0 + rk0, acc1 + rk1)


def _router(x, Wr, br):
    return pl.pallas_call(
        _router_body,
        grid=(NB,),
        in_specs=[
            pl.BlockSpec((RB, D), lambda i: (i, 0)),
            pl.BlockSpec((D, E), lambda i: (0, 0)),
            pl.BlockSpec((1, E), lambda i: (0, 0)),
        ],
        out_specs=[
            pl.BlockSpec((RB, 8), lambda i: (i, 0)),
            pl.BlockSpec((N, K), lambda i: (0, 0)),
            pl.BlockSpec((2, 128), lambda i: (0, 0)),
        ],
        out_shape=[
            jax.ShapeDtypeStruct((N, 8), jnp.float32),
            jax.ShapeDtypeStruct((N, K), jnp.float32),
            jax.ShapeDtypeStruct((2, 128), jnp.float32),
        ],
        scratch_shapes=[pltpu.VMEM((1, E), jnp.float32),
                        pltpu.VMEM((N, 8), jnp.float32)],
    )(x, Wr, br.reshape(1, E))


def _sc_scatter_x_body(slot0_hbm, slot1_hbm, x_hbm, xg_hbm,
                       xv, slots_v, sem0, sem1):
    wid = lax.axis_index("s") * 2 + lax.axis_index("c")
    t0 = wid * TPW
    pltpu.sync_copy(slot0_hbm.at[pl.ds(t0, TPW)], slots_v.at[0])
    pltpu.sync_copy(slot1_hbm.at[pl.ds(t0, TPW)], slots_v.at[1])
    pltpu.sync_copy(x_hbm.at[pl.ds(t0, TPW)], xv)
    cp0 = pltpu.async_copy(xv, xg_hbm.at[slots_v.at[0]], sem0)
    cp1 = pltpu.async_copy(xv, xg_hbm.at[slots_v.at[1]], sem1)
    cp0.wait()
    cp1.wait()


NCHK = TPW // GC


def _sc_gather_y_body(slot0_hbm, slot1_hbm, yg_hbm, y0_hbm, y1_hbm,
                      slots_v, y0v, y1v, sem00, sem01, sem10, sem11):
    wid = lax.axis_index("s") * 2 + lax.axis_index("c")
    t0 = wid * TPW
    pltpu.sync_copy(slot0_hbm.at[pl.ds(t0, TPW)], slots_v.at[0])
    pltpu.sync_copy(slot1_hbm.at[pl.ds(t0, TPW)], slots_v.at[1])
    sems0 = [sem00, sem01]
    sems1 = [sem10, sem11]

    def fire(c, b):
        cp0 = pltpu.async_copy(
            yg_hbm.at[slots_v.at[0, pl.ds(c * GC, GC)]], y0v.at[b], sems0[b])
        cp1 = pltpu.async_copy(
            yg_hbm.at[slots_v.at[1, pl.ds(c * GC, GC)]], y1v.at[b], sems1[b])
        return cp0, cp1

    pend = {}
    pend[0] = fire(0, 0)
    if NCHK > 1:
        pend[1] = fire(1, 1)
    for c in range(NCHK):
        b = c % 2
        cp0, cp1 = pend.pop(c)
        cp0.wait()
        cp1.wait()
        pltpu.sync_copy(y0v.at[b], y0_hbm.at[pl.ds(t0 + c * GC, GC)])
        pltpu.sync_copy(y1v.at[b], y1_hbm.at[pl.ds(t0 + c * GC, GC)])
        if c + 2 < NCHK:
            pend[c + 2] = fire(c + 2, b)


@functools.lru_cache(maxsize=None)
def _sc_kernels():
    mesh = plsc.VectorSubcoreMesh(core_axis_name="c", subcore_axis_name="s")
    scatter_x = functools.partial(
        pl.kernel, mesh=mesh,
        out_type=jax.ShapeDtypeStruct((P, D), jnp.float32),
        scratch_types=[
            pltpu.VMEM((TPW, D), jnp.float32),
            pltpu.VMEM((K, TPW), jnp.int32),
            pltpu.SemaphoreType.DMA,
            pltpu.SemaphoreType.DMA,
        ],
    )(_sc_scatter_x_body)
    gather_y = functools.partial(
        pl.kernel, mesh=mesh,
        out_type=[jax.ShapeDtypeStruct((N, D), jnp.float32),
                  jax.ShapeDtypeStruct((N, D), jnp.float32)],
        scratch_types=[
            pltpu.VMEM((K, TPW), jnp.int32),
            pltpu.VMEM((2, GC, D), jnp.float32),
            pltpu.VMEM((2, GC, D), jnp.float32),
            pltpu.SemaphoreType.DMA,
            pltpu.SemaphoreType.DMA,
            pltpu.SemaphoreType.DMA,
            pltpu.SemaphoreType.DMA,
        ],
    )(_sc_gather_y_body)
    return scatter_x, gather_y


def _ffn_body(te_ref, act_ref, xg_ref, w1_ref, b1_ref, w2_ref, b2_ref, yg_ref):
    i = pl.program_id(0)

    @pl.when(act_ref[i] == 1)
    def _compute():
        h = jnp.dot(xg_ref[...], w1_ref[0], preferred_element_type=jnp.float32)
        h = jnp.maximum(h + b1_ref[0], 0.0)
        y = jnp.dot(h, w2_ref[0], preferred_element_type=jnp.float32)
        yg_ref[...] = y + b2_ref[0]


def _grouped_ffn(te, act, xg, W1, b1, W2, b2):
    grid_spec = pltpu.PrefetchScalarGridSpec(
        num_scalar_prefetch=2,
        grid=(NT,),
        in_specs=[
            pl.BlockSpec((T, D), lambda i, te, act: (i, 0)),
            pl.BlockSpec((1, D, FF), lambda i, te, act: (te[i], 0, 0)),
            pl.BlockSpec((1, 1, FF), lambda i, te, act: (te[i], 0, 0)),
            pl.BlockSpec((1, FF, D), lambda i, te, act: (te[i], 0, 0)),
            pl.BlockSpec((1, 1, D), lambda i, te, act: (te[i], 0, 0)),
        ],
        out_specs=pl.BlockSpec((T, D), lambda i, te, act: (i, 0)),
    )
    return pl.pallas_call(
        _ffn_body,
        grid_spec=grid_spec,
        out_shape=jax.ShapeDtypeStruct((P, D), jnp.float32),
    )(te, act, xg, W1, b1.reshape(E, 1, FF), W2, b2.reshape(E, 1, D))


CB = 1024


def _combine_body(x_ref, r_ref, y0_ref, y1_ref, out_ref):
    col = lax.broadcasted_iota(jnp.int32, (CB, 8), 1)
    r = r_ref[...]
    w0 = jnp.sum(jnp.where(col == 2, r, 0.0), axis=1, keepdims=True)
    w1 = jnp.sum(jnp.where(col == 3, r, 0.0), axis=1, keepdims=True)
    out_ref[...] = x_ref[...] + w0 * y0_ref[...] + w1 * y1_ref[...]


def _combine(x, route, y0, y1):
    return pl.pallas_call(
        _combine_body,
        grid=(N // CB,),
        in_specs=[
            pl.BlockSpec((CB, D), lambda i: (i, 0)),
            pl.BlockSpec((CB, 8), lambda i: (i, 0)),
            pl.BlockSpec((CB, D), lambda i: (i, 0)),
            pl.BlockSpec((CB, D), lambda i: (i, 0)),
        ],
        out_specs=pl.BlockSpec((CB, D), lambda i: (i, 0)),
        out_shape=jax.ShapeDtypeStruct((N, D), jnp.float32),
    )(x, route, y0, y1)


def kernel(x, Wr, br, W1, b1, W2, b2):
    sc_scatter_x, sc_gather_y = _sc_kernels()
    route, slots, te_f = _router(x, Wr, br)
    te = te_f[0, :NT].astype(jnp.int32)
    act = te_f[1, :NT].astype(jnp.int32)
    st = slots.astype(jnp.int32).T
    slot0 = st[0]
    slot1 = st[1]
    xg = sc_scatter_x(slot0, slot1, x)
    yg = _grouped_ffn(te, act, xg, W1, b1, W2, b2)
    y0, y1 = sc_gather_y(slot0, slot1, yg)
    return _combine(x, route, y0, y1)


# speedup vs baseline: 1.0717x; 1.0062x over previous
import functools

import jax
import jax.numpy as jnp
from jax import lax
from jax.experimental import pallas as pl
from jax.experimental.pallas import tpu as pltpu
from jax.experimental.pallas import tpu_sc as plsc

N = 4096
D = 768
E = 16
FF = 1024
K = 2

T = 512
NT = (N * K) // T + E
P = NT * T
RB = 512
NB = N // RB

NW = 32
TPW = N // NW
SPW = P // NW
GC = 32


def _router_body(x_ref, wr_ref, br_ref, route_ref, slots_ref, te_ref,
                 carry, rbuf):
    i = pl.program_id(0)

    @pl.when(i == 0)
    def _init():
        carry[...] = jnp.zeros((1, E), jnp.float32)

    x = x_ref[...]
    logits = jnp.dot(x, wr_ref[...], preferred_element_type=jnp.float32)
    logits = logits + br_ref[...]
    lane = lax.broadcasted_iota(jnp.int32, (RB, E), 1)
    m1 = jnp.max(logits, axis=1, keepdims=True)
    idx1 = jnp.min(jnp.where(logits == m1, lane, E), axis=1, keepdims=True)
    l2 = jnp.where(lane == idx1, -jnp.inf, logits)
    m2 = jnp.max(l2, axis=1, keepdims=True)
    idx2 = jnp.min(jnp.where(l2 == m2, lane, E), axis=1, keepdims=True)
    b = jnp.exp(m2 - m1)
    w0 = 1.0 / (1.0 + b)
    w1 = b / (1.0 + b)

    oh0 = (lane == idx1).astype(jnp.float32)
    oh1 = (lane == idx2).astype(jnp.float32)
    ri = lax.broadcasted_iota(jnp.int32, (RB, RB), 0)
    ci = lax.broadcasted_iota(jnp.int32, (RB, RB), 1)
    ts = (ci < ri).astype(jnp.float32)
    c0 = carry[...]
    p0 = jnp.dot(ts, oh0, preferred_element_type=jnp.float32) + c0
    rank0 = jnp.sum(p0 * oh0, axis=1, keepdims=True)
    c1 = c0 + jnp.sum(oh0, axis=0, keepdims=True)
    p1 = jnp.dot(ts, oh1, preferred_element_type=jnp.float32) + c1
    rank1 = jnp.sum(p1 * oh1, axis=1, keepdims=True)
    c2 = c1 + jnp.sum(oh1, axis=0, keepdims=True)
    carry[...] = c2

    col = lax.broadcasted_iota(jnp.int32, (RB, 8), 1)
    route = jnp.where(col == 0, idx1.astype(jnp.float32), 0.0)
    route = route + jnp.where(col == 1, idx2.astype(jnp.float32), 0.0)
    route = route + jnp.where(col == 2, w0, 0.0)
    route = route + jnp.where(col == 3, w1, 0.0)
    route = route + jnp.where(col == 4, rank0, 0.0)
    route = route + jnp.where(col == 5, rank1, 0.0)
    route_ref[...] = route
    rbuf[pl.ds(i * RB, RB), :] = route

    @pl.when(i == NB - 1)
    def _finish():
        padded = jnp.floor((c2 + (T - 1)) * (1.0 / T)) * T
        ue = lax.broadcasted_iota(jnp.int32, (E, E), 0)
        uc = lax.broadcasted_iota(jnp.int32, (E, E), 1)
        ustrict = (ue < uc).astype(jnp.float32)
        base = jnp.dot(padded, ustrict,
                       preferred_element_type=jnp.float32)

        ti = lax.broadcasted_iota(jnp.int32, (1, 128), 1).astype(jnp.float32) * T
        acc = jnp.zeros((1, 128), jnp.float32)
        for e in range(E):
            acc = acc + (ti >= base[0:1, e:e + 1]).astype(jnp.float32)
        pend = base[0:1, E - 1:E] + padded[0:1, E - 1:E]
        act = (ti < pend).astype(jnp.float32)
        rowi = lax.broadcasted_iota(jnp.int32, (2, 128), 0)
        te_ref[...] = jnp.where(rowi == 0, acc - 1.0, act)

        colf = lax.broadcasted_iota(jnp.int32, (RB, 8), 1)
        col2 = lax.broadcasted_iota(jnp.int32, (RB, K), 1)
        lanef = lax.broadcasted_iota(jnp.int32, (RB, E), 1).astype(jnp.float32)
        for j in range(NB):
            r = rbuf[pl.ds(j * RB, RB), :]
            key0 = jnp.sum(jnp.where(colf == 0, r, 0.0), axis=1, keepdims=True)
            key1 = jnp.sum(jnp.where(colf == 1, r, 0.0), axis=1, keepdims=True)
            rk0 = jnp.sum(jnp.where(colf == 4, r, 0.0), axis=1, keepdims=True)
            rk1 = jnp.sum(jnp.where(colf == 5, r, 0.0), axis=1, keepdims=True)
            acc0 = jnp.sum(jnp.where(lanef == key0, base, 0.0),
                           axis=1, keepdims=True)
            acc1 = jnp.sum(jnp.where(lanef == key1, base, 0.0),
                           axis=1, keepdims=True)
            slots_ref[pl.ds(j * RB, RB), :] = jnp.where(
                col2 == 0, acc0 + rk0, acc1 + rk1)


def _router(x, Wr, br):
    return pl.pallas_call(
        _router_body,
        grid=(NB,),
        in_specs=[
            pl.BlockSpec((RB, D), lambda i: (i, 0)),
            pl.BlockSpec((D, E), lambda i: (0, 0)),
            pl.BlockSpec((1, E), lambda i: (0, 0)),
        ],
        out_specs=[
            pl.BlockSpec((RB, 8), lambda i: (i, 0)),
            pl.BlockSpec((N, K), lambda i: (0, 0)),
            pl.BlockSpec((2, 128), lambda i: (0, 0)),
        ],
        out_shape=[
            jax.ShapeDtypeStruct((N, 8), jnp.float32),
            jax.ShapeDtypeStruct((N, K), jnp.float32),
            jax.ShapeDtypeStruct((2, 128), jnp.float32),
        ],
        scratch_shapes=[pltpu.VMEM((1, E), jnp.float32),
                        pltpu.VMEM((N, 8), jnp.float32)],
    )(x, Wr, br.reshape(1, E))


XC = TPW // 2


def _sc_scatter_x_body(slot0_hbm, slot1_hbm, x_hbm, xg_hbm,
                       xv, slots_v, semx, sem0, sem1):
    wid = lax.axis_index("s") * 2 + lax.axis_index("c")
    t0 = wid * TPW
    cpx0 = pltpu.async_copy(x_hbm.at[pl.ds(t0, XC)], xv.at[0], semx)
    pltpu.sync_copy(slot0_hbm.at[pl.ds(t0, XC)], slots_v.at[0])
    pltpu.sync_copy(slot1_hbm.at[pl.ds(t0, XC)], slots_v.at[1])
    pltpu.sync_copy(slot0_hbm.at[pl.ds(t0 + XC, XC)], slots_v.at[2])
    pltpu.sync_copy(slot1_hbm.at[pl.ds(t0 + XC, XC)], slots_v.at[3])
    cpx1 = pltpu.async_copy(x_hbm.at[pl.ds(t0 + XC, XC)], xv.at[1], semx)
    cpx0.wait()
    cp00 = pltpu.async_copy(xv.at[0], xg_hbm.at[slots_v.at[0]], sem0)
    cp10 = pltpu.async_copy(xv.at[0], xg_hbm.at[slots_v.at[1]], sem1)
    cpx1.wait()
    cp01 = pltpu.async_copy(xv.at[1], xg_hbm.at[slots_v.at[2]], sem0)
    cp11 = pltpu.async_copy(xv.at[1], xg_hbm.at[slots_v.at[3]], sem1)
    cp00.wait()
    cp10.wait()
    cp01.wait()
    cp11.wait()


NCHK = TPW // GC


def _sc_gather_y_body(slot0_hbm, slot1_hbm, yg_hbm, y0_hbm, y1_hbm,
                      slots_v, y0v, y1v, sem00, sem01, sem10, sem11):
    wid = lax.axis_index("s") * 2 + lax.axis_index("c")
    t0 = wid * TPW
    pltpu.sync_copy(slot0_hbm.at[pl.ds(t0, TPW)], slots_v.at[0])
    pltpu.sync_copy(slot1_hbm.at[pl.ds(t0, TPW)], slots_v.at[1])
    sems0 = [sem00, sem01]
    sems1 = [sem10, sem11]

    def fire(c, b):
        cp0 = pltpu.async_copy(
            yg_hbm.at[slots_v.at[0, pl.ds(c * GC, GC)]], y0v.at[b], sems0[b])
        cp1 = pltpu.async_copy(
            yg_hbm.at[slots_v.at[1, pl.ds(c * GC, GC)]], y1v.at[b], sems1[b])
        return cp0, cp1

    pend = {}
    pend[0] = fire(0, 0)
    if NCHK > 1:
        pend[1] = fire(1, 1)
    for c in range(NCHK):
        b = c % 2
        cp0, cp1 = pend.pop(c)
        cp0.wait()
        cp1.wait()
        pltpu.sync_copy(y0v.at[b], y0_hbm.at[pl.ds(t0 + c * GC, GC)])
        pltpu.sync_copy(y1v.at[b], y1_hbm.at[pl.ds(t0 + c * GC, GC)])
        if c + 2 < NCHK:
            pend[c + 2] = fire(c + 2, b)


@functools.lru_cache(maxsize=None)
def _sc_kernels():
    mesh = plsc.VectorSubcoreMesh(core_axis_name="c", subcore_axis_name="s")
    scatter_x = functools.partial(
        pl.kernel, mesh=mesh,
        out_type=jax.ShapeDtypeStruct((P, D), jnp.float32),
        scratch_types=[
            pltpu.VMEM((2, XC, D), jnp.float32),
            pltpu.VMEM((2 * K, XC), jnp.int32),
            pltpu.SemaphoreType.DMA,
            pltpu.SemaphoreType.DMA,
            pltpu.SemaphoreType.DMA,
        ],
    )(_sc_scatter_x_body)
    gather_y = functools.partial(
        pl.kernel, mesh=mesh,
        out_type=[jax.ShapeDtypeStruct((N, D), jnp.float32),
                  jax.ShapeDtypeStruct((N, D), jnp.float32)],
        scratch_types=[
            pltpu.VMEM((K, TPW), jnp.int32),
            pltpu.VMEM((2, GC, D), jnp.float32),
            pltpu.VMEM((2, GC, D), jnp.float32),
            pltpu.SemaphoreType.DMA,
            pltpu.SemaphoreType.DMA,
            pltpu.SemaphoreType.DMA,
            pltpu.SemaphoreType.DMA,
        ],
    )(_sc_gather_y_body)
    return scatter_x, gather_y


def _ffn_body(te_ref, act_ref, xg_ref, w1_ref, b1_ref, w2_ref, b2_ref, yg_ref):
    i = pl.program_id(0)

    @pl.when(act_ref[i] == 1)
    def _compute():
        h = jnp.dot(xg_ref[...], w1_ref[0], preferred_element_type=jnp.float32)
        h = jnp.maximum(h + b1_ref[0], 0.0)
        y = jnp.dot(h, w2_ref[0], preferred_element_type=jnp.float32)
        yg_ref[...] = y + b2_ref[0]


def _grouped_ffn(te, act, xg, W1, b1, W2, b2):
    grid_spec = pltpu.PrefetchScalarGridSpec(
        num_scalar_prefetch=2,
        grid=(NT,),
        in_specs=[
            pl.BlockSpec((T, D), lambda i, te, act: (i, 0)),
            pl.BlockSpec((1, D, FF), lambda i, te, act: (te[i], 0, 0)),
            pl.BlockSpec((1, 1, FF), lambda i, te, act: (te[i], 0, 0)),
            pl.BlockSpec((1, FF, D), lambda i, te, act: (te[i], 0, 0)),
            pl.BlockSpec((1, 1, D), lambda i, te, act: (te[i], 0, 0)),
        ],
        out_specs=pl.BlockSpec((T, D), lambda i, te, act: (i, 0)),
    )
    return pl.pallas_call(
        _ffn_body,
        grid_spec=grid_spec,
        out_shape=jax.ShapeDtypeStruct((P, D), jnp.float32),
    )(te, act, xg, W1, b1.reshape(E, 1, FF), W2, b2.reshape(E, 1, D))


CB = 1024


def _combine_body(x_ref, r_ref, y0_ref, y1_ref, out_ref):
    col = lax.broadcasted_iota(jnp.int32, (CB, 8), 1)
    r = r_ref[...]
    w0 = jnp.sum(jnp.where(col == 2, r, 0.0), axis=1, keepdims=True)
    w1 = jnp.sum(jnp.where(col == 3, r, 0.0), axis=1, keepdims=True)
    out_ref[...] = x_ref[...] + w0 * y0_ref[...] + w1 * y1_ref[...]


def _combine(x, route, y0, y1):
    return pl.pallas_call(
        _combine_body,
        grid=(N // CB,),
        in_specs=[
            pl.BlockSpec((CB, D), lambda i: (i, 0)),
            pl.BlockSpec((CB, 8), lambda i: (i, 0)),
            pl.BlockSpec((CB, D), lambda i: (i, 0)),
            pl.BlockSpec((CB, D), lambda i: (i, 0)),
        ],
        out_specs=pl.BlockSpec((CB, D), lambda i: (i, 0)),
        out_shape=jax.ShapeDtypeStruct((N, D), jnp.float32),
    )(x, route, y0, y1)


def kernel(x, Wr, br, W1, b1, W2, b2):
    sc_scatter_x, sc_gather_y = _sc_kernels()
    route, slots, te_f = _router(x, Wr, br)
    te = te_f[0, :NT].astype(jnp.int32)
    act = te_f[1, :NT].astype(jnp.int32)
    st = slots.astype(jnp.int32).T
    slot0 = st[0]
    slot1 = st[1]
    xg = sc_scatter_x(slot0, slot1, x)
    yg = _grouped_ffn(te, act, xg, W1, b1, W2, b2)
    y0, y1 = sc_gather_y(slot0, slot1, yg)
    return _combine(x, route, y0, y1)


# speedup vs baseline: 1.0734x; 1.0016x over previous
import functools

import jax
import jax.numpy as jnp
from jax import lax
from jax.experimental import pallas as pl
from jax.experimental.pallas import tpu as pltpu
from jax.experimental.pallas import tpu_sc as plsc

N = 4096
D = 768
E = 16
FF = 1024
K = 2

T = 512
NT = (N * K) // T + E
P = NT * T
RB = 512
NB = N // RB

NW = 32
TPW = N // NW
GC = 32


def _router_body(x_ref, wr_ref, br_ref, route_ref, slots_ref, te_ref,
                 carry, rbuf):
    i = pl.program_id(0)

    @pl.when(i == 0)
    def _init():
        carry[...] = jnp.zeros((1, E), jnp.float32)

    x = x_ref[...]
    logits = jnp.dot(x, wr_ref[...], preferred_element_type=jnp.float32)
    logits = logits + br_ref[...]
    lane = lax.broadcasted_iota(jnp.int32, (RB, E), 1)
    m1 = jnp.max(logits, axis=1, keepdims=True)
    idx1 = jnp.min(jnp.where(logits == m1, lane, E), axis=1, keepdims=True)
    l2 = jnp.where(lane == idx1, -jnp.inf, logits)
    m2 = jnp.max(l2, axis=1, keepdims=True)
    idx2 = jnp.min(jnp.where(l2 == m2, lane, E), axis=1, keepdims=True)
    b = jnp.exp(m2 - m1)
    w0 = 1.0 / (1.0 + b)
    w1 = b / (1.0 + b)

    oh0 = (lane == idx1).astype(jnp.float32)
    oh1 = (lane == idx2).astype(jnp.float32)
    ri = lax.broadcasted_iota(jnp.int32, (RB, RB), 0)
    ci = lax.broadcasted_iota(jnp.int32, (RB, RB), 1)
    ts = (ci < ri).astype(jnp.float32)
    c0 = carry[...]
    p0 = jnp.dot(ts, oh0, preferred_element_type=jnp.float32) + c0
    rank0 = jnp.sum(p0 * oh0, axis=1, keepdims=True)
    c1 = c0 + jnp.sum(oh0, axis=0, keepdims=True)
    p1 = jnp.dot(ts, oh1, preferred_element_type=jnp.float32) + c1
    rank1 = jnp.sum(p1 * oh1, axis=1, keepdims=True)
    c2 = c1 + jnp.sum(oh1, axis=0, keepdims=True)
    carry[...] = c2

    col = lax.broadcasted_iota(jnp.int32, (RB, 8), 1)
    route = jnp.where(col == 0, idx1.astype(jnp.float32), 0.0)
    route = route + jnp.where(col == 1, idx2.astype(jnp.float32), 0.0)
    route = route + jnp.where(col == 2, w0, 0.0)
    route = route + jnp.where(col == 3, w1, 0.0)
    route = route + jnp.where(col == 4, rank0, 0.0)
    route = route + jnp.where(col == 5, rank1, 0.0)
    route_ref[...] = route
    rbuf[pl.ds(i * RB, RB), :] = route

    @pl.when(i == NB - 1)
    def _finish():
        padded = jnp.floor((c2 + (T - 1)) * (1.0 / T)) * T
        ue = lax.broadcasted_iota(jnp.int32, (E, E), 0)
        uc = lax.broadcasted_iota(jnp.int32, (E, E), 1)
        ustrict = (ue < uc).astype(jnp.float32)
        base = jnp.dot(padded, ustrict,
                       preferred_element_type=jnp.float32)

        ti = lax.broadcasted_iota(jnp.int32, (1, 128), 1).astype(jnp.float32) * T
        acc = jnp.zeros((1, 128), jnp.float32)
        for e in range(E):
            acc = acc + (ti >= base[0:1, e:e + 1]).astype(jnp.float32)
        pend = base[0:1, E - 1:E] + padded[0:1, E - 1:E]
        act = (ti < pend).astype(jnp.float32)
        rowi = lax.broadcasted_iota(jnp.int32, (2, 128), 0)
        te_ref[...] = jnp.where(rowi == 0, acc - 1.0, act)

        colf = lax.broadcasted_iota(jnp.int32, (RB, 8), 1)
        col2 = lax.broadcasted_iota(jnp.int32, (RB, K), 1)
        lanef = lax.broadcasted_iota(jnp.int32, (RB, E), 1).astype(jnp.float32)
        for j in range(NB):
            r = rbuf[pl.ds(j * RB, RB), :]
            key0 = jnp.sum(jnp.where(colf == 0, r, 0.0), axis=1, keepdims=True)
            key1 = jnp.sum(jnp.where(colf == 1, r, 0.0), axis=1, keepdims=True)
            rk0 = jnp.sum(jnp.where(colf == 4, r, 0.0), axis=1, keepdims=True)
            rk1 = jnp.sum(jnp.where(colf == 5, r, 0.0), axis=1, keepdims=True)
            acc0 = jnp.sum(jnp.where(lanef == key0, base, 0.0),
                           axis=1, keepdims=True)
            acc1 = jnp.sum(jnp.where(lanef == key1, base, 0.0),
                           axis=1, keepdims=True)
            slots_ref[pl.ds(j * RB, RB), :] = jnp.where(
                col2 == 0, acc0 + rk0, acc1 + rk1)


def _router(x, Wr, br):
    return pl.pallas_call(
        _router_body,
        grid=(NB,),
        in_specs=[
            pl.BlockSpec((RB, D), lambda i: (i, 0)),
            pl.BlockSpec((D, E), lambda i: (0, 0)),
            pl.BlockSpec((1, E), lambda i: (0, 0)),
        ],
        out_specs=[
            pl.BlockSpec((RB, 8), lambda i: (i, 0)),
            pl.BlockSpec((N, K), lambda i: (0, 0)),
            pl.BlockSpec((2, 128), lambda i: (0, 0)),
        ],
        out_shape=[
            jax.ShapeDtypeStruct((N, 8), jnp.float32),
            jax.ShapeDtypeStruct((N, K), jnp.float32),
            jax.ShapeDtypeStruct((2, 128), jnp.float32),
        ],
        scratch_shapes=[pltpu.VMEM((1, E), jnp.float32),
                        pltpu.VMEM((N, 8), jnp.float32)],
    )(x, Wr, br.reshape(1, E))


XC = TPW // 2


def _sc_scatter_x_body(slot0_hbm, slot1_hbm, x_hbm, xg_hbm,
                       xv, slots_v, semx, sem0, sem1):
    wid = lax.axis_index("s") * 2 + lax.axis_index("c")
    t0 = wid * TPW
    cpx0 = pltpu.async_copy(x_hbm.at[pl.ds(t0, XC)], xv.at[0], semx)
    pltpu.sync_copy(slot0_hbm.at[pl.ds(t0, XC)], slots_v.at[0])
    pltpu.sync_copy(slot1_hbm.at[pl.ds(t0, XC)], slots_v.at[1])
    pltpu.sync_copy(slot0_hbm.at[pl.ds(t0 + XC, XC)], slots_v.at[2])
    pltpu.sync_copy(slot1_hbm.at[pl.ds(t0 + XC, XC)], slots_v.at[3])
    cpx1 = pltpu.async_copy(x_hbm.at[pl.ds(t0 + XC, XC)], xv.at[1], semx)
    cpx0.wait()
    cp00 = pltpu.async_copy(xv.at[0], xg_hbm.at[slots_v.at[0]], sem0)
    cp10 = pltpu.async_copy(xv.at[0], xg_hbm.at[slots_v.at[1]], sem1)
    cpx1.wait()
    cp01 = pltpu.async_copy(xv.at[1], xg_hbm.at[slots_v.at[2]], sem0)
    cp11 = pltpu.async_copy(xv.at[1], xg_hbm.at[slots_v.at[3]], sem1)
    cp00.wait()
    cp10.wait()
    cp01.wait()
    cp11.wait()


NCHK = TPW // GC


def _sc_gather_y_body(slot0_hbm, slot1_hbm, yg_hbm, y0_hbm, y1_hbm,
                      slots_v, y0v, y1v, sem00, sem01, sem10, sem11):
    wid = lax.axis_index("s") * 2 + lax.axis_index("c")
    t0 = wid * TPW
    pltpu.sync_copy(slot0_hbm.at[pl.ds(t0, TPW)], slots_v.at[0])
    pltpu.sync_copy(slot1_hbm.at[pl.ds(t0, TPW)], slots_v.at[1])
    sems0 = [sem00, sem01]
    sems1 = [sem10, sem11]

    def fire(c, b):
        cp0 = pltpu.async_copy(
            yg_hbm.at[slots_v.at[0, pl.ds(c * GC, GC)]], y0v.at[b], sems0[b])
        cp1 = pltpu.async_copy(
            yg_hbm.at[slots_v.at[1, pl.ds(c * GC, GC)]], y1v.at[b], sems1[b])
        return cp0, cp1

    pend = {}
    pend[0] = fire(0, 0)
    if NCHK > 1:
        pend[1] = fire(1, 1)
    for c in range(NCHK):
        b = c % 2
        cp0, cp1 = pend.pop(c)
        cp0.wait()
        cp1.wait()
        pltpu.sync_copy(y0v.at[b], y0_hbm.at[pl.ds(t0 + c * GC, GC)])
        pltpu.sync_copy(y1v.at[b], y1_hbm.at[pl.ds(t0 + c * GC, GC)])
        if c + 2 < NCHK:
            pend[c + 2] = fire(c + 2, b)


@functools.lru_cache(maxsize=None)
def _sc_kernels():
    mesh = plsc.VectorSubcoreMesh(core_axis_name="c", subcore_axis_name="s")
    scatter_x = functools.partial(
        pl.kernel, mesh=mesh,
        out_type=jax.ShapeDtypeStruct((P, D), jnp.float32),
        scratch_types=[
            pltpu.VMEM((2, XC, D), jnp.float32),
            pltpu.VMEM((2 * K, XC), jnp.int32),
            pltpu.SemaphoreType.DMA,
            pltpu.SemaphoreType.DMA,
            pltpu.SemaphoreType.DMA,
        ],
    )(_sc_scatter_x_body)
    gather_y = functools.partial(
        pl.kernel, mesh=mesh,
        out_type=[jax.ShapeDtypeStruct((N, D), jnp.float32),
                  jax.ShapeDtypeStruct((N, D), jnp.float32)],
        scratch_types=[
            pltpu.VMEM((K, TPW), jnp.int32),
            pltpu.VMEM((2, GC, D), jnp.float32),
            pltpu.VMEM((2, GC, D), jnp.float32),
            pltpu.SemaphoreType.DMA,
            pltpu.SemaphoreType.DMA,
            pltpu.SemaphoreType.DMA,
            pltpu.SemaphoreType.DMA,
        ],
    )(_sc_gather_y_body)
    return scatter_x, gather_y


def _ffn_body(te_ref, act_ref, xg_ref, w1_ref, b1_ref, w2_ref, b2_ref, yg_ref):
    i = pl.program_id(0)

    @pl.when(act_ref[i] == 1)
    def _compute():
        h = jnp.dot(xg_ref[...], w1_ref[0], preferred_element_type=jnp.float32)
        h = jnp.maximum(h + b1_ref[0], 0.0)
        y = jnp.dot(h, w2_ref[0], preferred_element_type=jnp.float32)
        yg_ref[...] = y + b2_ref[0]


def _grouped_ffn(te, act, xg, W1, b1, W2, b2):
    grid_spec = pltpu.PrefetchScalarGridSpec(
        num_scalar_prefetch=2,
        grid=(NT,),
        in_specs=[
            pl.BlockSpec((T, D), lambda i, te, act: (i, 0)),
            pl.BlockSpec((1, D, FF), lambda i, te, act: (te[i], 0, 0)),
            pl.BlockSpec((1, 1, FF), lambda i, te, act: (te[i], 0, 0)),
            pl.BlockSpec((1, FF, D), lambda i, te, act: (te[i], 0, 0)),
            pl.BlockSpec((1, 1, D), lambda i, te, act: (te[i], 0, 0)),
        ],
        out_specs=pl.BlockSpec((T, D), lambda i, te, act: (i, 0)),
    )
    return pl.pallas_call(
        _ffn_body,
        grid_spec=grid_spec,
        out_shape=jax.ShapeDtypeStruct((P, D), jnp.float32),
    )(te, act, xg, W1, b1.reshape(E, 1, FF), W2, b2.reshape(E, 1, D))


CB = 1024


def _combine_body(x_ref, r_ref, y0_ref, y1_ref, out_ref):
    col = lax.broadcasted_iota(jnp.int32, (CB, 8), 1)
    r = r_ref[...]
    w0 = jnp.sum(jnp.where(col == 2, r, 0.0), axis=1, keepdims=True)
    w1 = jnp.sum(jnp.where(col == 3, r, 0.0), axis=1, keepdims=True)
    out_ref[...] = x_ref[...] + w0 * y0_ref[...] + w1 * y1_ref[...]


def _combine(x, route, y0, y1):
    return pl.pallas_call(
        _combine_body,
        grid=(N // CB,),
        in_specs=[
            pl.BlockSpec((CB, D), lambda i: (i, 0)),
            pl.BlockSpec((CB, 8), lambda i: (i, 0)),
            pl.BlockSpec((CB, D), lambda i: (i, 0)),
            pl.BlockSpec((CB, D), lambda i: (i, 0)),
        ],
        out_specs=pl.BlockSpec((CB, D), lambda i: (i, 0)),
        out_shape=jax.ShapeDtypeStruct((N, D), jnp.float32),
    )(x, route, y0, y1)


def kernel(x, Wr, br, W1, b1, W2, b2):
    sc_scatter_x, sc_gather_y = _sc_kernels()
    route, slots, te_f = _router(x, Wr, br)
    te = te_f[0, :NT].astype(jnp.int32)
    act = te_f[1, :NT].astype(jnp.int32)
    st = slots.astype(jnp.int32).T
    slot0 = st[0]
    slot1 = st[1]
    xg = sc_scatter_x(slot0, slot1, x)
    yg = _grouped_ffn(te, act, xg, W1, b1, W2, b2)
    y0, y1 = sc_gather_y(slot0, slot1, yg)
    return _combine(x, route, y0, y1)
